# baseline (speedup 1.0000x reference)
import functools

import jax
import jax.numpy as jnp
from jax import lax
from jax.experimental import pallas as pl
from jax.experimental.pallas import tpu as pltpu
from jax.experimental.pallas import tpu_sc as plsc

_ROWS = 64
_COLS = 32768
_N = _ROWS * _COLS
_K = max(50, _N // 100)

_NW = 32
_B = 2048
_RB = 8
_CB = 8192
_MASK31 = 0x7FFFFFFF

_mesh = plsc.VectorSubcoreMesh(
    core_axis_name="c", subcore_axis_name="s", num_cores=2, num_subcores=16
)


def _to_key(v):
    s = lax.bitcast_convert_type(v, jnp.int32)
    return jnp.where(s < 0, s ^ jnp.int32(_MASK31), s)


@functools.partial(
    pl.kernel,
    mesh=_mesh,
    out_type=jax.ShapeDtypeStruct((2, _B), jnp.int32),
    scratch_types=[
        pltpu.VMEM((_RB, _CB), jnp.float32),
        pltpu.VMEM((16 * _B,), jnp.int32),
        pltpu.VMEM((_B,), jnp.int32),
        pltpu.VMEM((16, 128), jnp.int32),
        pltpu.VMEM((128,), jnp.int32),
        pltpu.VMEM_SHARED((16, _B), jnp.int32),
        pltpu.SemaphoreType.DMA,
    ],
    compiler_params=pltpu.CompilerParams(needs_layout_passes=False),
)
def _sc_hist1(data_hbm, out_h1, buf, hist2d, hist, tmp, seg, shared, sem):
    scid = lax.axis_index("c")
    sid = lax.axis_index("s")
    wid = scid * 16 + sid
    rb = wid // 4
    cb = wid % 4
    cp = pltpu.async_copy(
        data_hbm.at[pl.ds(rb * _RB, _RB), pl.ds(cb * _CB, _CB)], buf, sem
    )

    zero16 = jnp.zeros((16,), jnp.int32)
    ones16 = jnp.ones((16,), jnp.int32)
    lane_base = lax.iota(jnp.int32, 16) * _B + jnp.int32(1024)

    @plsc.parallel_loop(0, 16 * _B // 16, unroll=8)
    def _(i):
        hist2d[pl.ds(i * 16, 16)] = zero16

    cp.wait()

    for r in range(_RB):

        @plsc.parallel_loop(0, _CB // 16, unroll=8)
        def _(i):
            key = _to_key(buf[r, pl.ds(i * 16, 16)])
            b = lax.shift_right_arithmetic(key, 21)
            plsc.addupdate_scatter(hist2d, [lane_base + b], ones16)

    @plsc.parallel_loop(0, _B // 16, unroll=4)
    def _(i):
        acc = hist2d[pl.ds(i * 16, 16)]
        for j in range(1, 16):
            acc = acc + hist2d[pl.ds(j * _B + i * 16, 16)]
        hist[pl.ds(i * 16, 16)] = acc

    pltpu.sync_copy(hist, shared.at[sid])
    plsc.subcore_barrier()
    pltpu.sync_copy(shared.at[:, pl.ds(sid * 128, 128)], tmp)

    @plsc.parallel_loop(0, 8, unroll=4)
    def _(c):
        acc = tmp[0, pl.ds(c * 16, 16)]
        for j in range(1, 16):
            acc = acc + tmp[j, pl.ds(c * 16, 16)]
        seg[pl.ds(c * 16, 16)] = acc

    pltpu.sync_copy(seg, out_h1.at[scid, pl.ds(sid * 128, 128)])


@functools.partial(
    pl.kernel,
    mesh=_mesh,
    out_type=(
        jax.ShapeDtypeStruct((_NW, _B), jnp.int32),
        jax.ShapeDtypeStruct((_NW, _B), jnp.float32),
        jax.ShapeDtypeStruct((_NW, 16), jnp.float32),
    ),
    scratch_types=[
        pltpu.VMEM((2, _CB), jnp.float32),
        pltpu.VMEM((2, _CB), jnp.float32),
        pltpu.VMEM((16 * _B,), jnp.int32),
        pltpu.VMEM((16 * _B,), jnp.float32),
        pltpu.VMEM((_B,), jnp.int32),
        pltpu.VMEM((_B,), jnp.float32),
        pltpu.VMEM((2, _B), jnp.int32),
        pltpu.VMEM((16,), jnp.float32),
        pltpu.SemaphoreType.DMA,
        pltpu.SemaphoreType.DMA,
    ],
    compiler_params=pltpu.CompilerParams(needs_layout_passes=False),
)
def _sc_hist2(data_hbm, h1_hbm, out_ch, out_sh, out_sacc,
              buf0, buf1, hist2d, histf2d, hist, histf, h1v, svec,
              sem0, sem1):
    scid = lax.axis_index("c")
    sid = lax.axis_index("s")
    wid = scid * 16 + sid
    rb = wid // 4
    cb = wid % 4
    bufs = (buf0, buf1)
    sems = (sem0, sem1)

    def _start(q, buf, sem):
        return pltpu.async_copy(
            data_hbm.at[pl.ds(rb * _RB + q * 2, 2), pl.ds(cb * _CB, _CB)],
            buf,
            sem,
        )

    cp0 = _start(0, buf0, sem0)
    pltpu.sync_copy(h1_hbm, h1v)

    zero16 = jnp.zeros((16,), jnp.int32)
    zero16f = jnp.zeros((16,), jnp.float32)
    ones16 = jnp.ones((16,), jnp.int32)
    lane_base = lax.iota(jnp.int32, 16) * _B

    @plsc.parallel_loop(0, 16 * _B // 16, unroll=8)
    def _(i):
        hist2d[pl.ds(i * 16, 16)] = zero16
        histf2d[pl.ds(i * 16, 16)] = zero16f

    def scan_step(i, st):
        cstar, sprime, run = st
        c = 127 - i
        chunk = h1v[0, pl.ds(c * 16, 16)] + h1v[1, pl.ds(c * 16, 16)]
        t = lax.reduce_sum_p.bind(chunk, axes=(0,))
        run2 = run + t
        found = jnp.logical_and(run2 >= _K, cstar < 0)
        cstar = jnp.where(found, c, cstar)
        sprime = jnp.where(found, run, sprime)
        return cstar, sprime, run2

    cstar, sprime, _ = lax.fori_loop(
        0, 128, scan_step, (jnp.int32(-1), jnp.int32(0), jnp.int32(0))
    )
    chunk = (
        h1v[0, pl.ds(cstar * 16, 16)] + h1v[1, pl.ds(cstar * 16, 16)]
    )
    cs = plsc.cumsum(lax.rev(chunk, (0,)))
    istar = plsc.all_reduce_ffs(cs + sprime >= _K)
    b1v = cstar * 16 + 15 - istar - jnp.int32(1024)

    accs = (zero16f,) * 8
    cps = [cp0, None]
    for q in range(4):
        if q < 3:
            cps[(q + 1) % 2] = _start(q + 1, bufs[(q + 1) % 2], sems[(q + 1) % 2])
        cps[q % 2].wait()
        buf = bufs[q % 2]
        for r in range(2):

            @plsc.parallel_loop(0, _CB // 16, step=8, carry=accs)
            def accs(i, accs):
                out = []
                for u in range(8):
                    v = buf[r, pl.ds((i + u) * 16, 16)]
                    key = _to_key(v)
                    bsig = lax.shift_right_arithmetic(key, 21)
                    a = accs[u] + jnp.where(bsig > b1v, v, jnp.float32(0.0))
                    inb = bsig == b1v
                    b2 = lax.shift_right_arithmetic(key, 10) & jnp.int32(0x7FF)
                    idx = lane_base + b2
                    plsc.addupdate_scatter(hist2d, [idx], ones16, mask=inb)
                    plsc.addupdate_scatter(histf2d, [idx], v, mask=inb)
                    out.append(a)
                return tuple(out)

    sacc = accs[0]
    for u in range(1, 8):
        sacc = sacc + accs[u]
    svec[...] = sacc

    @plsc.parallel_loop(0, _B // 16, unroll=4)
    def _(i):
        acc = hist2d[pl.ds(i * 16, 16)]
        accf = histf2d[pl.ds(i * 16, 16)]
        for j in range(1, 16):
            acc = acc + hist2d[pl.ds(j * _B + i * 16, 16)]
            accf = accf + histf2d[pl.ds(j * _B + i * 16, 16)]
        hist[pl.ds(i * 16, 16)] = acc
        histf[pl.ds(i * 16, 16)] = accf

    pltpu.sync_copy(hist, out_ch.at[wid])
    pltpu.sync_copy(histf, out_sh.at[wid])
    pltpu.sync_copy(svec, out_sacc.at[wid])


def _bisect_high(h, bid, base_count):
    def step(i, p):
        q = p + (jnp.int32(1) << (10 - i))
        f = base_count + jnp.sum(jnp.where(bid >= q, h, 0))
        return jnp.where(f >= _K, q, p)

    return lax.fori_loop(0, 11, step, jnp.int32(0))


def _tc_final_body(h1_ref, ch_ref, sh_ref, sacc_ref, out_ref):
    h1 = jnp.sum(h1_ref[...], axis=0)
    h = jnp.sum(ch_ref[...], axis=0)
    f = jnp.sum(sh_ref[...], axis=0)
    s_above1 = jnp.sum(sacc_ref[...])
    bid = (
        lax.broadcasted_iota(jnp.int32, (16, 128), 0) * 128
        + lax.broadcasted_iota(jnp.int32, (16, 128), 1)
    )
    p1 = _bisect_high(h1, bid, jnp.int32(0))
    c_above1 = jnp.sum(jnp.where(bid > p1, h1, 0))
    b1s = p1 - jnp.int32(1024)
    p = _bisect_high(h, bid, c_above1)
    c_above2 = c_above1 + jnp.sum(jnp.where(bid > p, h, 0))
    s_above2 = s_above1 + jnp.sum(jnp.where(bid > p, f, jnp.float32(0.0)))
    key_mid = (b1s * jnp.int32(2048) + p) * jnp.int32(1024) + jnp.int32(512)
    t_bits = jnp.where(key_mid < 0, key_mid ^ jnp.int32(_MASK31), key_mid)
    t = lax.bitcast_convert_type(t_bits, jnp.float32)
    out_ref[0, 0] = s_above2 + (jnp.int32(_K) - c_above2).astype(jnp.float32) * t


def kernel(data):
    h1 = _sc_hist1(data)
    ch2, sh2, sacc = _sc_hist2(data, h1)
    out = pl.pallas_call(
        _tc_final_body,
        out_shape=jax.ShapeDtypeStruct((1, 1), jnp.float32),
        in_specs=[
            pl.BlockSpec(memory_space=pltpu.VMEM),
            pl.BlockSpec(memory_space=pltpu.VMEM),
            pl.BlockSpec(memory_space=pltpu.VMEM),
            pl.BlockSpec(memory_space=pltpu.VMEM),
        ],
        out_specs=pl.BlockSpec(memory_space=pltpu.SMEM),
    )(h1.reshape(2, 16, 128), ch2.reshape(_NW, 16, 128),
      sh2.reshape(_NW, 16, 128), sacc)
    return out[0, 0]


# speedup vs baseline: 1.0581x; 1.0581x over previous
import functools

import jax
import jax.numpy as jnp
from jax import lax
from jax.experimental import pallas as pl
from jax.experimental.pallas import tpu as pltpu
from jax.experimental.pallas import tpu_sc as plsc

_ROWS = 64
_COLS = 32768
_N = _ROWS * _COLS
_K = max(50, _N // 100)

_NW = 32
_B = 2048
_RB = 8
_CB = 8192
_MASK31 = 0x7FFFFFFF

_mesh = plsc.VectorSubcoreMesh(
    core_axis_name="c", subcore_axis_name="s", num_cores=2, num_subcores=16
)


def _to_key(v):
    s = lax.bitcast_convert_type(v, jnp.int32)
    return jnp.where(s < 0, s ^ jnp.int32(_MASK31), s)


@functools.partial(
    pl.kernel,
    mesh=_mesh,
    out_type=jax.ShapeDtypeStruct((2, _B), jnp.int32),
    scratch_types=[
        pltpu.VMEM((_RB, _CB), jnp.float32),
        pltpu.VMEM((16 * _B,), jnp.int32),
        pltpu.VMEM((_B,), jnp.int32),
        pltpu.VMEM((16, 128), jnp.int32),
        pltpu.VMEM((128,), jnp.int32),
        pltpu.VMEM_SHARED((16, _B), jnp.int32),
        pltpu.SemaphoreType.DMA,
    ],
    compiler_params=pltpu.CompilerParams(needs_layout_passes=False),
)
def _sc_hist1(data_hbm, out_h1, buf, hist2d, hist, tmp, seg, shared, sem):
    scid = lax.axis_index("c")
    sid = lax.axis_index("s")
    wid = scid * 16 + sid
    rb = wid // 4
    cb = wid % 4
    cp = pltpu.async_copy(
        data_hbm.at[pl.ds(rb * _RB, _RB), pl.ds(cb * _CB, _CB)], buf, sem
    )

    zero16 = jnp.zeros((16,), jnp.int32)
    ones16 = jnp.ones((16,), jnp.int32)
    lane_base = lax.iota(jnp.int32, 16) * _B + jnp.int32(1024)

    @plsc.parallel_loop(0, 16 * _B // 16, unroll=8)
    def _(i):
        hist2d[pl.ds(i * 16, 16)] = zero16

    cp.wait()

    for r in range(_RB):

        @plsc.parallel_loop(0, _CB // 16, unroll=8)
        def _(i):
            key = _to_key(buf[r, pl.ds(i * 16, 16)])
            b = lax.shift_right_arithmetic(key, 21)
            plsc.addupdate_scatter(hist2d, [lane_base + b], ones16)

    @plsc.parallel_loop(0, _B // 16, unroll=4)
    def _(i):
        acc = hist2d[pl.ds(i * 16, 16)]
        for j in range(1, 16):
            acc = acc + hist2d[pl.ds(j * _B + i * 16, 16)]
        hist[pl.ds(i * 16, 16)] = acc

    pltpu.sync_copy(hist, shared.at[sid])
    plsc.subcore_barrier()
    pltpu.sync_copy(shared.at[:, pl.ds(sid * 128, 128)], tmp)

    @plsc.parallel_loop(0, 8, unroll=4)
    def _(c):
        acc = tmp[0, pl.ds(c * 16, 16)]
        for j in range(1, 16):
            acc = acc + tmp[j, pl.ds(c * 16, 16)]
        seg[pl.ds(c * 16, 16)] = acc

    pltpu.sync_copy(seg, out_h1.at[scid, pl.ds(sid * 128, 128)])


@functools.partial(
    pl.kernel,
    mesh=_mesh,
    out_type=(
        jax.ShapeDtypeStruct((_NW, _B), jnp.int32),
        jax.ShapeDtypeStruct((_NW, _B), jnp.float32),
        jax.ShapeDtypeStruct((_NW, 16), jnp.float32),
    ),
    scratch_types=[
        pltpu.VMEM((2, _CB), jnp.float32),
        pltpu.VMEM((2, _CB), jnp.float32),
        pltpu.VMEM((16 * _B,), jnp.int32),
        pltpu.VMEM((16 * _B,), jnp.float32),
        pltpu.VMEM((_B,), jnp.int32),
        pltpu.VMEM((_B,), jnp.float32),
        pltpu.VMEM((2, _B), jnp.int32),
        pltpu.VMEM((16,), jnp.float32),
        pltpu.SemaphoreType.DMA,
        pltpu.SemaphoreType.DMA,
    ],
    compiler_params=pltpu.CompilerParams(needs_layout_passes=False),
)
def _sc_hist2(data_hbm, h1_hbm, out_ch, out_sh, out_sacc,
              buf0, buf1, hist2d, histf2d, hist, histf, h1v, svec,
              sem0, sem1):
    scid = lax.axis_index("c")
    sid = lax.axis_index("s")
    wid = scid * 16 + sid
    rb = wid // 4
    cb = wid % 4
    bufs = (buf0, buf1)
    sems = (sem0, sem1)

    def _start(q, buf, sem):
        return pltpu.async_copy(
            data_hbm.at[pl.ds(rb * _RB + q * 2, 2), pl.ds(cb * _CB, _CB)],
            buf,
            sem,
        )

    cp0 = _start(0, buf0, sem0)
    pltpu.sync_copy(h1_hbm, h1v)

    zero16 = jnp.zeros((16,), jnp.int32)
    zero16f = jnp.zeros((16,), jnp.float32)
    ones16 = jnp.ones((16,), jnp.int32)
    lane_base = lax.iota(jnp.int32, 16) * _B

    @plsc.parallel_loop(0, 16 * _B // 16, unroll=8)
    def _(i):
        hist2d[pl.ds(i * 16, 16)] = zero16
        histf2d[pl.ds(i * 16, 16)] = zero16f

    def scan_step(i, st):
        cstar, sprime, run = st
        c = 127 - i
        chunk = h1v[0, pl.ds(c * 16, 16)] + h1v[1, pl.ds(c * 16, 16)]
        t = lax.reduce_sum_p.bind(chunk, axes=(0,))
        run2 = run + t
        found = jnp.logical_and(run2 >= _K, cstar < 0)
        cstar = jnp.where(found, c, cstar)
        sprime = jnp.where(found, run, sprime)
        return cstar, sprime, run2

    cstar, sprime, _ = lax.fori_loop(
        0, 128, scan_step, (jnp.int32(-1), jnp.int32(0), jnp.int32(0))
    )
    chunk = (
        h1v[0, pl.ds(cstar * 16, 16)] + h1v[1, pl.ds(cstar * 16, 16)]
    )
    cs = plsc.cumsum(lax.rev(chunk, (0,)))
    istar = plsc.all_reduce_ffs(cs + sprime >= _K)
    b1v = cstar * 16 + 15 - istar - jnp.int32(1024)

    accs = (zero16f,) * 8
    cps = [cp0, None]
    for q in range(4):
        if q < 3:
            cps[(q + 1) % 2] = _start(q + 1, bufs[(q + 1) % 2], sems[(q + 1) % 2])
        cps[q % 2].wait()
        buf = bufs[q % 2]
        for r in range(2):

            @plsc.parallel_loop(0, _CB // 16, step=8, carry=accs)
            def accs(i, accs):
                out = []
                for u in range(8):
                    v = buf[r, pl.ds((i + u) * 16, 16)]
                    key = _to_key(v)
                    bsig = lax.shift_right_arithmetic(key, 21)
                    a = accs[u] + jnp.where(bsig > b1v, v, jnp.float32(0.0))
                    inb = bsig == b1v
                    b2 = lax.shift_right_arithmetic(key, 10) & jnp.int32(0x7FF)
                    idx = lane_base + b2
                    plsc.addupdate_scatter(hist2d, [idx], ones16, mask=inb)
                    plsc.addupdate_scatter(histf2d, [idx], v, mask=inb)
                    out.append(a)
                return tuple(out)

    sacc = accs[0]
    for u in range(1, 8):
        sacc = sacc + accs[u]
    svec[...] = sacc

    @plsc.parallel_loop(0, _B // 16, unroll=4)
    def _(i):
        acc = hist2d[pl.ds(i * 16, 16)]
        accf = histf2d[pl.ds(i * 16, 16)]
        for j in range(1, 16):
            acc = acc + hist2d[pl.ds(j * _B + i * 16, 16)]
            accf = accf + histf2d[pl.ds(j * _B + i * 16, 16)]
        hist[pl.ds(i * 16, 16)] = acc
        histf[pl.ds(i * 16, 16)] = accf

    pltpu.sync_copy(hist, out_ch.at[wid])
    pltpu.sync_copy(histf, out_sh.at[wid])
    pltpu.sync_copy(svec, out_sacc.at[wid])


def _bisect_high(h, bid, base_count):
    p = jnp.int32(0)
    for i in range(11):
        q = p + (jnp.int32(1) << (10 - i))
        f = base_count + jnp.sum(jnp.where(bid >= q, h, 0))
        p = jnp.where(f >= _K, q, p)
    return p


def _tc_final_body(h1_ref, ch_ref, sh_ref, sacc_ref, out_ref):
    h1 = jnp.sum(h1_ref[...], axis=0, keepdims=True)
    h = jnp.sum(ch_ref[...], axis=0, keepdims=True)
    f = jnp.sum(sh_ref[...], axis=0, keepdims=True)
    s_above1 = jnp.sum(sacc_ref[...])
    bid = lax.broadcasted_iota(jnp.int32, (1, _B), 1)
    p1 = _bisect_high(h1, bid, jnp.int32(0))
    c_above1 = jnp.sum(jnp.where(bid > p1, h1, 0))
    b1s = p1 - jnp.int32(1024)
    p = _bisect_high(h, bid, c_above1)
    c_above2 = c_above1 + jnp.sum(jnp.where(bid > p, h, 0))
    s_above2 = s_above1 + jnp.sum(jnp.where(bid > p, f, jnp.float32(0.0)))
    key_mid = (b1s * jnp.int32(2048) + p) * jnp.int32(1024) + jnp.int32(512)
    t_bits = jnp.where(key_mid < 0, key_mid ^ jnp.int32(_MASK31), key_mid)
    t = lax.bitcast_convert_type(t_bits, jnp.float32)
    out_ref[0, 0] = s_above2 + (jnp.int32(_K) - c_above2).astype(jnp.float32) * t


def kernel(data):
    h1 = _sc_hist1(data)
    ch2, sh2, sacc = _sc_hist2(data, h1)
    out = pl.pallas_call(
        _tc_final_body,
        out_shape=jax.ShapeDtypeStruct((1, 1), jnp.float32),
        in_specs=[
            pl.BlockSpec(memory_space=pltpu.VMEM),
            pl.BlockSpec(memory_space=pltpu.VMEM),
            pl.BlockSpec(memory_space=pltpu.VMEM),
            pl.BlockSpec(memory_space=pltpu.VMEM),
        ],
        out_specs=pl.BlockSpec(memory_space=pltpu.SMEM),
    )(h1, ch2, sh2, sacc)
    return out[0, 0]


# speedup vs baseline: 1.0729x; 1.0141x over previous
import functools

import jax
import jax.numpy as jnp
from jax import lax
from jax.experimental import pallas as pl
from jax.experimental.pallas import tpu as pltpu
from jax.experimental.pallas import tpu_sc as plsc

_ROWS = 64
_COLS = 32768
_N = _ROWS * _COLS
_K = max(50, _N // 100)

_NW = 32
_B = 2048
_RB = 8
_CB = 8192
_MASK31 = 0x7FFFFFFF

_mesh = plsc.VectorSubcoreMesh(
    core_axis_name="c", subcore_axis_name="s", num_cores=2, num_subcores=16
)


def _to_key(v):
    s = lax.bitcast_convert_type(v, jnp.int32)
    return jnp.where(s < 0, s ^ jnp.int32(_MASK31), s)


@functools.partial(
    pl.kernel,
    mesh=_mesh,
    out_type=jax.ShapeDtypeStruct((2, _B), jnp.int32),
    scratch_types=[
        pltpu.VMEM((2, _CB), jnp.float32),
        pltpu.VMEM((2, _CB), jnp.float32),
        pltpu.VMEM((16 * _B,), jnp.int32),
        pltpu.VMEM((_B,), jnp.int32),
        pltpu.VMEM((16, 128), jnp.int32),
        pltpu.VMEM((128,), jnp.int32),
        pltpu.VMEM_SHARED((16, _B), jnp.int32),
        pltpu.SemaphoreType.DMA,
        pltpu.SemaphoreType.DMA,
    ],
    compiler_params=pltpu.CompilerParams(needs_layout_passes=False),
)
def _sc_hist1(data_hbm, out_h1, buf0, buf1, hist2d, hist, tmp, seg, shared,
              sem0, sem1):
    scid = lax.axis_index("c")
    sid = lax.axis_index("s")
    wid = scid * 16 + sid
    rb = wid // 4
    cb = wid % 4
    bufs = (buf0, buf1)
    sems = (sem0, sem1)

    def _start(q, buf, sem):
        return pltpu.async_copy(
            data_hbm.at[pl.ds(rb * _RB + q * 2, 2), pl.ds(cb * _CB, _CB)],
            buf,
            sem,
        )

    cp0 = _start(0, buf0, sem0)

    zero16 = jnp.zeros((16,), jnp.int32)
    ones16 = jnp.ones((16,), jnp.int32)
    lane_base = lax.iota(jnp.int32, 16) * _B + jnp.int32(1024)

    @plsc.parallel_loop(0, 16 * _B // 16, unroll=8)
    def _(i):
        hist2d[pl.ds(i * 16, 16)] = zero16

    cps = [cp0, None]
    for q in range(4):
        if q < 3:
            cps[(q + 1) % 2] = _start(q + 1, bufs[(q + 1) % 2], sems[(q + 1) % 2])
        cps[q % 2].wait()
        buf = bufs[q % 2]
        for r in range(2):

            @plsc.parallel_loop(0, _CB // 16, unroll=8)
            def _(i):
                key = _to_key(buf[r, pl.ds(i * 16, 16)])
                b = lax.shift_right_arithmetic(key, 21)
                plsc.addupdate_scatter(hist2d, [lane_base + b], ones16)

    @plsc.parallel_loop(0, _B // 16, unroll=4)
    def _(i):
        acc = hist2d[pl.ds(i * 16, 16)]
        for j in range(1, 16):
            acc = acc + hist2d[pl.ds(j * _B + i * 16, 16)]
        hist[pl.ds(i * 16, 16)] = acc

    pltpu.sync_copy(hist, shared.at[sid])
    plsc.subcore_barrier()
    pltpu.sync_copy(shared.at[:, pl.ds(sid * 128, 128)], tmp)

    @plsc.parallel_loop(0, 8, unroll=4)
    def _(c):
        acc = tmp[0, pl.ds(c * 16, 16)]
        for j in range(1, 16):
            acc = acc + tmp[j, pl.ds(c * 16, 16)]
        seg[pl.ds(c * 16, 16)] = acc

    pltpu.sync_copy(seg, out_h1.at[scid, pl.ds(sid * 128, 128)])


@functools.partial(
    pl.kernel,
    mesh=_mesh,
    out_type=(
        jax.ShapeDtypeStruct((_NW, _B), jnp.int32),
        jax.ShapeDtypeStruct((_NW, _B), jnp.float32),
        jax.ShapeDtypeStruct((_NW, 16), jnp.float32),
    ),
    scratch_types=[
        pltpu.VMEM((2, _CB), jnp.float32),
        pltpu.VMEM((2, _CB), jnp.float32),
        pltpu.VMEM((16 * _B,), jnp.int32),
        pltpu.VMEM((16 * _B,), jnp.float32),
        pltpu.VMEM((_B,), jnp.int32),
        pltpu.VMEM((_B,), jnp.float32),
        pltpu.VMEM((2, _B), jnp.int32),
        pltpu.VMEM((16,), jnp.float32),
        pltpu.SemaphoreType.DMA,
        pltpu.SemaphoreType.DMA,
    ],
    compiler_params=pltpu.CompilerParams(needs_layout_passes=False),
)
def _sc_hist2(data_hbm, h1_hbm, out_ch, out_sh, out_sacc,
              buf0, buf1, hist2d, histf2d, hist, histf, h1v, svec,
              sem0, sem1):
    scid = lax.axis_index("c")
    sid = lax.axis_index("s")
    wid = scid * 16 + sid
    rb = wid // 4
    cb = wid % 4
    bufs = (buf0, buf1)
    sems = (sem0, sem1)

    def _start(q, buf, sem):
        return pltpu.async_copy(
            data_hbm.at[pl.ds(rb * _RB + q * 2, 2), pl.ds(cb * _CB, _CB)],
            buf,
            sem,
        )

    cp0 = _start(0, buf0, sem0)
    pltpu.sync_copy(h1_hbm, h1v)

    zero16 = jnp.zeros((16,), jnp.int32)
    zero16f = jnp.zeros((16,), jnp.float32)
    ones16 = jnp.ones((16,), jnp.int32)
    lane_base = lax.iota(jnp.int32, 16) * _B

    @plsc.parallel_loop(0, 16 * _B // 16, unroll=8)
    def _(i):
        hist2d[pl.ds(i * 16, 16)] = zero16
        histf2d[pl.ds(i * 16, 16)] = zero16f

    def scan_step(i, st):
        cstar, sprime, run = st
        c = 127 - i
        chunk = h1v[0, pl.ds(c * 16, 16)] + h1v[1, pl.ds(c * 16, 16)]
        t = lax.reduce_sum_p.bind(chunk, axes=(0,))
        run2 = run + t
        found = jnp.logical_and(run2 >= _K, cstar < 0)
        cstar = jnp.where(found, c, cstar)
        sprime = jnp.where(found, run, sprime)
        return cstar, sprime, run2

    cstar, sprime, _ = lax.fori_loop(
        0, 128, scan_step, (jnp.int32(-1), jnp.int32(0), jnp.int32(0))
    )
    chunk = (
        h1v[0, pl.ds(cstar * 16, 16)] + h1v[1, pl.ds(cstar * 16, 16)]
    )
    cs = plsc.cumsum(lax.rev(chunk, (0,)))
    istar = plsc.all_reduce_ffs(cs + sprime >= _K)
    b1v = cstar * 16 + 15 - istar - jnp.int32(1024)

    accs = (zero16f,) * 8
    cps = [cp0, None]
    for q in range(4):
        if q < 3:
            cps[(q + 1) % 2] = _start(q + 1, bufs[(q + 1) % 2], sems[(q + 1) % 2])
        cps[q % 2].wait()
        buf = bufs[q % 2]
        for r in range(2):

            @plsc.parallel_loop(0, _CB // 16, step=8, carry=accs)
            def accs(i, accs):
                out = []
                for u in range(8):
                    v = buf[r, pl.ds((i + u) * 16, 16)]
                    key = _to_key(v)
                    bsig = lax.shift_right_arithmetic(key, 21)
                    a = accs[u] + jnp.where(bsig > b1v, v, jnp.float32(0.0))
                    inb = bsig == b1v
                    b2 = lax.shift_right_arithmetic(key, 10) & jnp.int32(0x7FF)
                    idx = lane_base + b2
                    plsc.addupdate_scatter(hist2d, [idx], ones16, mask=inb)
                    plsc.addupdate_scatter(histf2d, [idx], v, mask=inb)
                    out.append(a)
                return tuple(out)

    sacc = accs[0]
    for u in range(1, 8):
        sacc = sacc + accs[u]
    svec[...] = sacc

    @plsc.parallel_loop(0, _B // 16, unroll=4)
    def _(i):
        acc = hist2d[pl.ds(i * 16, 16)]
        accf = histf2d[pl.ds(i * 16, 16)]
        for j in range(1, 16):
            acc = acc + hist2d[pl.ds(j * _B + i * 16, 16)]
            accf = accf + histf2d[pl.ds(j * _B + i * 16, 16)]
        hist[pl.ds(i * 16, 16)] = acc
        histf[pl.ds(i * 16, 16)] = accf

    pltpu.sync_copy(hist, out_ch.at[wid])
    pltpu.sync_copy(histf, out_sh.at[wid])
    pltpu.sync_copy(svec, out_sacc.at[wid])


def _bisect_high(h, bid, base_count):
    p = jnp.int32(0)
    for i in range(11):
        q = p + (jnp.int32(1) << (10 - i))
        f = base_count + jnp.sum(jnp.where(bid >= q, h, 0))
        p = jnp.where(f >= _K, q, p)
    return p


def _tc_final_body(h1_ref, ch_ref, sh_ref, sacc_ref, out_ref):
    h1 = jnp.sum(h1_ref[...], axis=0, keepdims=True)
    h = jnp.sum(ch_ref[...], axis=0, keepdims=True)
    f = jnp.sum(sh_ref[...], axis=0, keepdims=True)
    s_above1 = jnp.sum(sacc_ref[...])
    bid = lax.broadcasted_iota(jnp.int32, (1, _B), 1)
    p1 = _bisect_high(h1, bid, jnp.int32(0))
    c_above1 = jnp.sum(jnp.where(bid > p1, h1, 0))
    b1s = p1 - jnp.int32(1024)
    p = _bisect_high(h, bid, c_above1)
    c_above2 = c_above1 + jnp.sum(jnp.where(bid > p, h, 0))
    s_above2 = s_above1 + jnp.sum(jnp.where(bid > p, f, jnp.float32(0.0)))
    key_mid = (b1s * jnp.int32(2048) + p) * jnp.int32(1024) + jnp.int32(512)
    t_bits = jnp.where(key_mid < 0, key_mid ^ jnp.int32(_MASK31), key_mid)
    t = lax.bitcast_convert_type(t_bits, jnp.float32)
    out_ref[0, 0] = s_above2 + (jnp.int32(_K) - c_above2).astype(jnp.float32) * t


def kernel(data):
    h1 = _sc_hist1(data)
    ch2, sh2, sacc = _sc_hist2(data, h1)
    out = pl.pallas_call(
        _tc_final_body,
        out_shape=jax.ShapeDtypeStruct((1, 1), jnp.float32),
        in_specs=[
            pl.BlockSpec(memory_space=pltpu.VMEM),
            pl.BlockSpec(memory_space=pltpu.VMEM),
            pl.BlockSpec(memory_space=pltpu.VMEM),
            pl.BlockSpec(memory_space=pltpu.VMEM),
        ],
        out_specs=pl.BlockSpec(memory_space=pltpu.SMEM),
    )(h1, ch2, sh2, sacc)
    return out[0, 0]


# speedup vs baseline: 1.1619x; 1.0830x over previous
import functools

import jax
import jax.numpy as jnp
from jax import lax
from jax.experimental import pallas as pl
from jax.experimental.pallas import tpu as pltpu
from jax.experimental.pallas import tpu_sc as plsc

_ROWS = 64
_COLS = 32768
_N = _ROWS * _COLS
_K = max(50, _N // 100)

_NW = 32
_B = 2048
_B2 = _B + 16
_RB = 8
_CB = 8192
_MASK31 = 0x7FFFFFFF

_mesh = plsc.VectorSubcoreMesh(
    core_axis_name="c", subcore_axis_name="s", num_cores=2, num_subcores=16
)


def _to_key(v):
    s = lax.bitcast_convert_type(v, jnp.int32)
    return jnp.where(s < 0, s ^ jnp.int32(_MASK31), s)


@functools.partial(
    pl.kernel,
    mesh=_mesh,
    out_type=jax.ShapeDtypeStruct((2, _B), jnp.int32),
    scratch_types=[
        pltpu.VMEM((2, _CB), jnp.float32),
        pltpu.VMEM((2, _CB), jnp.float32),
        pltpu.VMEM((16 * _B,), jnp.int32),
        pltpu.VMEM((_B,), jnp.int32),
        pltpu.VMEM((16, 128), jnp.int32),
        pltpu.VMEM((128,), jnp.int32),
        pltpu.VMEM_SHARED((16, _B), jnp.int32),
        pltpu.SemaphoreType.DMA,
        pltpu.SemaphoreType.DMA,
    ],
    compiler_params=pltpu.CompilerParams(needs_layout_passes=False),
)
def _sc_hist1(data_hbm, out_h1, buf0, buf1, hist2d, hist, tmp, seg, shared,
              sem0, sem1):
    scid = lax.axis_index("c")
    sid = lax.axis_index("s")
    wid = scid * 16 + sid
    rb = wid // 4
    cb = wid % 4
    bufs = (buf0, buf1)
    sems = (sem0, sem1)

    def _start(q, buf, sem):
        return pltpu.async_copy(
            data_hbm.at[pl.ds(rb * _RB + q * 2, 2), pl.ds(cb * _CB, _CB)],
            buf,
            sem,
        )

    cp0 = _start(0, buf0, sem0)

    zero16 = jnp.zeros((16,), jnp.int32)
    ones16 = jnp.ones((16,), jnp.int32)
    lane_base = lax.iota(jnp.int32, 16) * _B + jnp.int32(1024)

    @plsc.parallel_loop(0, 16 * _B // 16, unroll=8)
    def _(i):
        hist2d[pl.ds(i * 16, 16)] = zero16

    cps = [cp0, None]
    for q in range(4):
        if q < 3:
            cps[(q + 1) % 2] = _start(q + 1, bufs[(q + 1) % 2], sems[(q + 1) % 2])
        cps[q % 2].wait()
        buf = bufs[q % 2]
        for r in range(2):

            @plsc.parallel_loop(0, _CB // 16, unroll=8)
            def _(i):
                key = _to_key(buf[r, pl.ds(i * 16, 16)])
                b = lax.shift_right_arithmetic(key, 21)
                plsc.addupdate_scatter(hist2d, [lane_base + b], ones16)

    @plsc.parallel_loop(0, _B // 16, unroll=4)
    def _(i):
        acc = hist2d[pl.ds(i * 16, 16)]
        for j in range(1, 16):
            acc = acc + hist2d[pl.ds(j * _B + i * 16, 16)]
        hist[pl.ds(i * 16, 16)] = acc

    pltpu.sync_copy(hist, shared.at[sid])
    plsc.subcore_barrier()
    pltpu.sync_copy(shared.at[:, pl.ds(sid * 128, 128)], tmp)

    @plsc.parallel_loop(0, 8, unroll=4)
    def _(c):
        acc = tmp[0, pl.ds(c * 16, 16)]
        for j in range(1, 16):
            acc = acc + tmp[j, pl.ds(c * 16, 16)]
        seg[pl.ds(c * 16, 16)] = acc

    pltpu.sync_copy(seg, out_h1.at[scid, pl.ds(sid * 128, 128)])


@functools.partial(
    pl.kernel,
    mesh=_mesh,
    out_type=(
        jax.ShapeDtypeStruct((_NW, _B), jnp.int32),
        jax.ShapeDtypeStruct((_NW, _B), jnp.float32),
        jax.ShapeDtypeStruct((_NW, 16), jnp.float32),
    ),
    scratch_types=[
        pltpu.VMEM((2, _CB), jnp.float32),
        pltpu.VMEM((2, _CB), jnp.float32),
        pltpu.VMEM((16 * _B2,), jnp.int32),
        pltpu.VMEM((16 * _B2,), jnp.float32),
        pltpu.VMEM((_B,), jnp.int32),
        pltpu.VMEM((_B,), jnp.float32),
        pltpu.VMEM((2, _B), jnp.int32),
        pltpu.VMEM((16,), jnp.float32),
        pltpu.SemaphoreType.DMA,
        pltpu.SemaphoreType.DMA,
    ],
    compiler_params=pltpu.CompilerParams(needs_layout_passes=False),
)
def _sc_hist2(data_hbm, h1_hbm, out_ch, out_sh, out_sacc,
              buf0, buf1, hist2d, histf2d, hist, histf, h1v, svec,
              sem0, sem1):
    scid = lax.axis_index("c")
    sid = lax.axis_index("s")
    wid = scid * 16 + sid
    rb = wid // 4
    cb = wid % 4
    bufs = (buf0, buf1)
    sems = (sem0, sem1)

    def _start(q, buf, sem):
        return pltpu.async_copy(
            data_hbm.at[pl.ds(rb * _RB + q * 2, 2), pl.ds(cb * _CB, _CB)],
            buf,
            sem,
        )

    cp0 = _start(0, buf0, sem0)
    pltpu.sync_copy(h1_hbm, h1v)

    zero16 = jnp.zeros((16,), jnp.int32)
    zero16f = jnp.zeros((16,), jnp.float32)
    ones16 = jnp.ones((16,), jnp.int32)
    lane_base = lax.iota(jnp.int32, 16) * _B2

    @plsc.parallel_loop(0, 16 * _B2 // 16, unroll=8)
    def _(i):
        hist2d[pl.ds(i * 16, 16)] = zero16
        histf2d[pl.ds(i * 16, 16)] = zero16f

    def scan_step(i, st):
        cstar, sprime, run = st
        c = 127 - i
        chunk = h1v[0, pl.ds(c * 16, 16)] + h1v[1, pl.ds(c * 16, 16)]
        t = lax.reduce_sum_p.bind(chunk, axes=(0,))
        run2 = run + t
        found = jnp.logical_and(run2 >= _K, cstar < 0)
        cstar = jnp.where(found, c, cstar)
        sprime = jnp.where(found, run, sprime)
        return cstar, sprime, run2

    cstar, sprime, _ = lax.fori_loop(
        0, 128, scan_step, (jnp.int32(-1), jnp.int32(0), jnp.int32(0))
    )
    chunk = (
        h1v[0, pl.ds(cstar * 16, 16)] + h1v[1, pl.ds(cstar * 16, 16)]
    )
    cs = plsc.cumsum(lax.rev(chunk, (0,)))
    istar = plsc.all_reduce_ffs(cs + sprime >= _K)
    b1v = cstar * 16 + 15 - istar - jnp.int32(1024)

    cps = [cp0, None]
    for q in range(4):
        if q < 3:
            cps[(q + 1) % 2] = _start(q + 1, bufs[(q + 1) % 2], sems[(q + 1) % 2])
        cps[q % 2].wait()
        buf = bufs[q % 2]
        for r in range(2):

            @plsc.parallel_loop(0, _CB // 16, unroll=8)
            def _(i):
                v = buf[r, pl.ds(i * 16, 16)]
                key = _to_key(v)
                bsig = lax.shift_right_arithmetic(key, 21)
                keep = bsig >= b1v
                inb = bsig == b1v
                b2 = lax.shift_right_arithmetic(key, 10) & jnp.int32(0x7FF)
                idx = lane_base + jnp.where(inb, b2, jnp.int32(_B))
                plsc.addupdate_scatter(hist2d, [idx], ones16, mask=keep)
                plsc.addupdate_scatter(histf2d, [idx], v, mask=keep)

    svec[...] = plsc.load_gather(
        histf2d, [lax.iota(jnp.int32, 16) * _B2 + jnp.int32(_B)]
    )

    @plsc.parallel_loop(0, _B // 16, unroll=4)
    def _(i):
        acc = hist2d[pl.ds(i * 16, 16)]
        accf = histf2d[pl.ds(i * 16, 16)]
        for j in range(1, 16):
            acc = acc + hist2d[pl.ds(j * _B2 + i * 16, 16)]
            accf = accf + histf2d[pl.ds(j * _B2 + i * 16, 16)]
        hist[pl.ds(i * 16, 16)] = acc
        histf[pl.ds(i * 16, 16)] = accf

    pltpu.sync_copy(hist, out_ch.at[wid])
    pltpu.sync_copy(histf, out_sh.at[wid])
    pltpu.sync_copy(svec, out_sacc.at[wid])


def _bisect_high(h, bid, base_count):
    p = jnp.int32(0)
    for i in range(11):
        q = p + (jnp.int32(1) << (10 - i))
        f = base_count + jnp.sum(jnp.where(bid >= q, h, 0))
        p = jnp.where(f >= _K, q, p)
    return p


def _tc_final_body(h1_ref, ch_ref, sh_ref, sacc_ref, out_ref):
    h1 = jnp.sum(h1_ref[...], axis=0, keepdims=True)
    h = jnp.sum(ch_ref[...], axis=0, keepdims=True)
    f = jnp.sum(sh_ref[...], axis=0, keepdims=True)
    s_above1 = jnp.sum(sacc_ref[...])
    bid = lax.broadcasted_iota(jnp.int32, (1, _B), 1)
    p1 = _bisect_high(h1, bid, jnp.int32(0))
    c_above1 = jnp.sum(jnp.where(bid > p1, h1, 0))
    b1s = p1 - jnp.int32(1024)
    p = _bisect_high(h, bid, c_above1)
    c_above2 = c_above1 + jnp.sum(jnp.where(bid > p, h, 0))
    s_above2 = s_above1 + jnp.sum(jnp.where(bid > p, f, jnp.float32(0.0)))
    key_mid = (b1s * jnp.int32(2048) + p) * jnp.int32(1024) + jnp.int32(512)
    t_bits = jnp.where(key_mid < 0, key_mid ^ jnp.int32(_MASK31), key_mid)
    t = lax.bitcast_convert_type(t_bits, jnp.float32)
    out_ref[0, 0] = s_above2 + (jnp.int32(_K) - c_above2).astype(jnp.float32) * t


def kernel(data):
    h1 = _sc_hist1(data)
    ch2, sh2, sacc = _sc_hist2(data, h1)
    out = pl.pallas_call(
        _tc_final_body,
        out_shape=jax.ShapeDtypeStruct((1, 1), jnp.float32),
        in_specs=[
            pl.BlockSpec(memory_space=pltpu.VMEM),
            pl.BlockSpec(memory_space=pltpu.VMEM),
            pl.BlockSpec(memory_space=pltpu.VMEM),
            pl.BlockSpec(memory_space=pltpu.VMEM),
        ],
        out_specs=pl.BlockSpec(memory_space=pltpu.SMEM),
    )(h1, ch2, sh2, sacc)
    return out[0, 0]
